# baseline (speedup 1.0000x reference)
import functools

import numpy as np
import jax
from jax import lax
import jax.numpy as jnp
from jax.experimental import pallas as pl
from jax.experimental.pallas import tpu as pltpu
from jax.experimental.pallas import tpu_sc as plsc

_KS0 = np.uint32(0)
_KS1 = np.uint32(1234)
_KS2 = np.uint32(0 ^ 1234 ^ 0x1BD11BDA)
_KSCH = (_KS0, _KS1, _KS2)
_ROT = ((13, 15, 26, 6), (17, 29, 16, 24))

_LO = np.nextafter(np.float32(-1.0), np.float32(0.0))

_ERFINV_COEF = (
    np.float32(7.9537160e-05), np.float32(-1.7974426e-03),
    np.float32(1.6159540e-02), np.float32(-7.1861416e-02),
    np.float32(1.5740238e-01), np.float32(-1.6248988e-01),
    np.float32(3.1472448e-01), np.float32(-1.9252552e-02),
    np.float32(1.2543312e+00),
)


def _threefry_bits(x1_init):
    x1 = x1_init
    x0 = x1
    first = True
    for i in range(5):
        for r in _ROT[i % 2]:
            if first:
                first = False
            else:
                x0 = x0 + x1
            x1 = (x1 << r) | (x1 >> (32 - r))
            x1 = x1 ^ x0
        x0 = x0 + _KSCH[(i + 1) % 3]
        x1 = x1 + np.uint32(int(_KSCH[(i + 2) % 3]) + i + 1)
    return x0 ^ x1


def _bits_to_normal(bits):
    fl2 = jax.lax.bitcast_convert_type(
        (bits >> 9) | np.uint32(0x40000000), jnp.float32)
    u = jnp.maximum(_LO, fl2 - np.float32(3.0))
    s = jnp.sqrt(-jnp.log2(np.float32(1.0) - u * u))
    p = jnp.full(s.shape, _ERFINV_COEF[0])
    for c in _ERFINV_COEF[1:]:
        p = p * s + c
    return p * u


def _sc_gather(table2, idx):
    info = plsc.get_sparse_core_info()
    nc = info.num_cores
    nb = idx.shape[0]
    d = table2.shape[1]
    mesh = plsc.VectorSubcoreMesh(core_axis_name="c", subcore_axis_name="s")

    @functools.partial(
        pl.kernel, mesh=mesh,
        out_type=jax.ShapeDtypeStruct((nb, d), jnp.float32),
        scratch_types=[
            pltpu.VMEM((nb,), jnp.int32),
            pltpu.VMEM((nb, d), jnp.float32),
            pltpu.SemaphoreType.DMA,
        ],
    )
    def k(table_hbm, idx_hbm, out_hbm, idx_v, rows_v, sem):
        wid = lax.axis_index("s") * nc + lax.axis_index("c")

        @pl.when(wid == 0)
        def _():
            pltpu.sync_copy(idx_hbm, idx_v)
            pltpu.async_copy(table_hbm.at[idx_v], rows_v, sem).wait()
            pltpu.sync_copy(rows_v, out_hbm)

    return k(table2, idx)


def _noiser_kernel(sa_ref, sm_ref, x0_ref, xt_ref, eps_ref,
                   *, nb_steps, c, w, h):
    b = pl.program_id(0)
    x0 = x0_ref[0]
    shp = (c, w, h)
    local = (jax.lax.broadcasted_iota(jnp.uint32, shp, 0) * np.uint32(w * h)
             + jax.lax.broadcasted_iota(jnp.uint32, shp, 1) * np.uint32(h)
             + jax.lax.broadcasted_iota(jnp.uint32, shp, 2))
    for s in range(nb_steps):
        base = (b * nb_steps + s) * (c * w * h) + 1234
        x1_init = jax.lax.convert_element_type(base, jnp.uint32) + local
        eps = _bits_to_normal(_threefry_bits(x1_init))
        sa = sa_ref[b, s]
        sm = sm_ref[b, s]
        eps_ref[0, s] = eps
        xt_ref[0, s] = sa * x0 + eps * sm


def kernel(x_0, t, sqrt_alphas_cum_prod, sqrt_minus_one_alphas_cum_prod):
    b, c, w, h = x_0.shape
    nb_steps = t.shape[1]

    table2 = jnp.concatenate(
        [sqrt_alphas_cum_prod[:, None], sqrt_minus_one_alphas_cum_prod[:, None],
         jnp.zeros((sqrt_alphas_cum_prod.shape[0], 126), jnp.float32)], axis=1)
    gathered = _sc_gather(table2, t.reshape(-1))
    sa_g = gathered[:, 0].reshape(b, nb_steps)
    sm_g = gathered[:, 1].reshape(b, nb_steps)

    out_shape = [
        jax.ShapeDtypeStruct((b, nb_steps, c, w, h), jnp.float32),
        jax.ShapeDtypeStruct((b, nb_steps, c, w, h), jnp.float32),
    ]
    kern = functools.partial(_noiser_kernel, nb_steps=nb_steps, c=c, w=w, h=h)
    xt, eps = pl.pallas_call(
        kern,
        grid=(b,),
        in_specs=[
            pl.BlockSpec(memory_space=pltpu.SMEM),
            pl.BlockSpec(memory_space=pltpu.SMEM),
            pl.BlockSpec((1, c, w, h), lambda bi: (bi, 0, 0, 0)),
        ],
        out_specs=[
            pl.BlockSpec((1, nb_steps, c, w, h),
                         lambda bi: (bi, 0, 0, 0, 0)),
            pl.BlockSpec((1, nb_steps, c, w, h),
                         lambda bi: (bi, 0, 0, 0, 0)),
        ],
        out_shape=out_shape,
        compiler_params=pltpu.CompilerParams(
            dimension_semantics=("arbitrary",)),
    )(sa_g, sm_g, x_0)
    return (xt, eps)
